# baseline (speedup 1.0000x reference)
import functools

import jax
import jax.numpy as jnp
from jax import lax
from jax.experimental import pallas as pl
from jax.experimental.pallas import tpu as pltpu
from jax.experimental.pallas import tpu_sc as plsc

N_NODE = 10000
D_FEAT = 128
N_ACC = 10112
ROWS_PER_TILE = N_ACC // 16
PAD_IDX = 10000
N_BATCH = 16
CHUNK = 128
N_WORKERS = 32
GRP = 8


def _pack_edges(edge_index, e_pad):
    e = edge_index.shape[1]
    cpt = e_pad // (N_WORKERS * CHUNK)
    padv = jnp.full((e_pad - e,), PAD_IDX, jnp.int32)
    src = jnp.concatenate([edge_index[0], padv]).reshape(N_WORKERS, cpt, CHUNK)
    dst = jnp.concatenate([edge_index[1], padv]).reshape(N_WORKERS, cpt, CHUNK)
    return jnp.concatenate([src, dst], axis=1)


def _sc_mesh():
    return plsc.VectorSubcoreMesh(core_axis_name="c", subcore_axis_name="s")


def _deg_body(cpt_sub, cpt_glob, eidx_sub, eidx_glob, out_hbm,
              idxd_s, idxd_g, ones_v, zbuf_v, deg_sub_sh, deg_glob_sh, sem):
    c = lax.axis_index("c")
    s = lax.axis_index("s")
    wid = c * 16 + s

    pltpu.sync_copy(eidx_sub.at[wid, pl.ds(cpt_sub, cpt_sub)], idxd_s)
    pltpu.sync_copy(eidx_glob.at[wid, pl.ds(cpt_glob, cpt_glob)], idxd_g)

    def fill(i, _):
        ones_v[pl.ds(i * 16, 16)] = jnp.ones((16,), jnp.float32)
        return 0
    lax.fori_loop(0, CHUNK // 16, fill, 0)

    def zfill(i, _):
        zbuf_v[pl.ds(i * 16, 16)] = jnp.zeros((16,), jnp.float32)
        return 0
    lax.fori_loop(0, N_ACC // 16, zfill, 0)

    @pl.when(s == 0)
    def _():
        pltpu.sync_copy(zbuf_v, deg_sub_sh)
        pltpu.sync_copy(zbuf_v, deg_glob_sh)
    plsc.subcore_barrier()

    def scatter_graph(idxd, deg_sh, cpt):
        def body(i, _):
            for j in range(8):
                pltpu.async_copy(ones_v, deg_sh.at[idxd.at[i * 8 + j]], sem,
                                 add=True)
            for j in range(8):
                pltpu.make_async_copy(ones_v, deg_sh.at[idxd.at[0]],
                                      sem).wait()
            return 0
        lax.fori_loop(0, cpt // 8, body, 0)

    scatter_graph(idxd_s, deg_sub_sh, cpt_sub)
    scatter_graph(idxd_g, deg_glob_sh, cpt_glob)
    plsc.subcore_barrier()

    @pl.when(jnp.logical_and(s == 0, c == 0))
    def _():
        pltpu.sync_copy(deg_sub_sh, out_hbm.at[0, 0])
        pltpu.sync_copy(deg_glob_sh, out_hbm.at[1, 0])

    @pl.when(jnp.logical_and(s == 0, c == 1))
    def _():
        pltpu.sync_copy(deg_sub_sh, out_hbm.at[0, 1])
        pltpu.sync_copy(deg_glob_sh, out_hbm.at[1, 1])


def _make_deg_kernel(cpt_sub, cpt_glob):
    return pl.kernel(
        functools.partial(_deg_body, cpt_sub, cpt_glob),
        out_type=jax.ShapeDtypeStruct((2, 2, N_ACC), jnp.float32),
        mesh=_sc_mesh(),
        scratch_types=[
            pltpu.VMEM((cpt_sub, CHUNK), jnp.int32),
            pltpu.VMEM((cpt_glob, CHUNK), jnp.int32),
            pltpu.VMEM((CHUNK,), jnp.float32),
            pltpu.VMEM((N_ACC,), jnp.float32),
            pltpu.VMEM_SHARED((N_ACC,), jnp.float32),
            pltpu.VMEM_SHARED((N_ACC,), jnp.float32),
            pltpu.SemaphoreType.DMA,
        ],
    )


def _rows_body(cpt, h_hbm, eidx_hbm, zeros_hbm, out_hbm,
               idx_s, idx_d, rows_v, acc_sh, gsem, ssem):
    c = lax.axis_index("c")
    s = lax.axis_index("s")
    wid = c * 16 + s

    pltpu.sync_copy(zeros_hbm, rows_v.at[0])
    base = s * ROWS_PER_TILE
    for j in range(4):
        pltpu.sync_copy(rows_v.at[0], acc_sh.at[pl.ds(base + j * CHUNK, CHUNK)])
    pltpu.sync_copy(rows_v.at[0, pl.ds(0, ROWS_PER_TILE - 4 * CHUNK)],
                    acc_sh.at[pl.ds(base + 4 * CHUNK, ROWS_PER_TILE - 4 * CHUNK)])
    plsc.subcore_barrier()

    def body(i, _):
        for b in range(2):
            k = 2 * i + b
            pltpu.sync_copy(eidx_hbm.at[wid, k], idx_s)
            pltpu.sync_copy(eidx_hbm.at[wid, cpt + k], idx_d.at[b])
            pltpu.async_copy(h_hbm.at[idx_s], rows_v.at[b], gsem.at[b])

            @pl.when(k >= 1)
            def _():
                pltpu.make_async_copy(rows_v.at[1 - b],
                                      acc_sh.at[idx_d.at[1 - b]],
                                      ssem.at[1 - b]).wait()
            pltpu.make_async_copy(h_hbm.at[idx_s], rows_v.at[b],
                                  gsem.at[b]).wait()
            pltpu.async_copy(rows_v.at[b], acc_sh.at[idx_d.at[b]],
                             ssem.at[b], add=True)
        return 0
    lax.fori_loop(0, cpt // 2, body, 0)
    pltpu.make_async_copy(rows_v.at[1], acc_sh.at[idx_d.at[1]],
                          ssem.at[1]).wait()
    plsc.subcore_barrier()

    sizes = [CHUNK] * 4 + [ROWS_PER_TILE - 4 * CHUNK]

    @pl.when(c == 0)
    def _():
        o = 0
        for sz in sizes:
            pltpu.sync_copy(acc_sh.at[pl.ds(base + o, sz)],
                            out_hbm.at[0, pl.ds(base + o, sz)])
            o += sz

    @pl.when(c == 1)
    def _():
        o = 0
        for sz in sizes:
            pltpu.sync_copy(acc_sh.at[pl.ds(base + o, sz)],
                            out_hbm.at[1, pl.ds(base + o, sz)])
            o += sz


def _make_rows_kernel(cpt):
    return pl.kernel(
        functools.partial(_rows_body, cpt),
        out_type=jax.ShapeDtypeStruct((2, N_ACC, D_FEAT), jnp.float32),
        mesh=_sc_mesh(),
        scratch_types=[
            pltpu.VMEM((CHUNK,), jnp.int32),
            pltpu.VMEM((2, CHUNK), jnp.int32),
            pltpu.VMEM((2, CHUNK, D_FEAT), jnp.float32),
            pltpu.VMEM_SHARED((N_ACC, D_FEAT), jnp.float32),
            pltpu.SemaphoreType.DMA((2,)),
            pltpu.SemaphoreType.DMA((2,)),
        ],
    )


def _dinv(degp_ref, g):
    deg = degp_ref[g, 0, 0:N_NODE, :] + degp_ref[g, 1, 0:N_NODE, :] + 1.0
    return lax.rsqrt(jnp.maximum(deg, 1e-12))


def _mm_sub_body(x_ref, w_ref, degp_ref, o_ref):
    h = jnp.dot(x_ref[:], w_ref[:], preferred_element_type=jnp.float32)
    o_ref[0:N_NODE, :] = h * _dinv(degp_ref, 0)
    o_ref[N_NODE:N_NODE + 8, :] = jnp.zeros((8, D_FEAT), jnp.float32)


def _fin_sub_body(hsub_ref, acc_ref, degp_ref, b_ref, batch_ref, o_ref):
    dinv = _dinv(degp_ref, 0)
    pre = (hsub_ref[0:N_NODE, :] + acc_ref[0, 0:N_NODE, :]
           + acc_ref[1, 0:N_NODE, :]) * dinv + b_ref[:]
    hs = jnp.maximum(pre, 0.0)
    onehot = (batch_ref[:] == lax.broadcasted_iota(
        jnp.int32, (N_NODE, N_BATCH), 1)).astype(jnp.float32)
    dn = (((0,), (0,)), ((), ()))
    psum = lax.dot_general(onehot, hs, dn,
                           preferred_element_type=jnp.float32)
    cnt = lax.dot_general(onehot, jnp.ones((N_NODE, 1), jnp.float32), dn,
                          preferred_element_type=jnp.float32)
    o_ref[:] = psum / jnp.maximum(cnt, 1.0)


def _mm_glob_body(x_ref, w_ref, degp_ref, pooled_ref, sidx_ref, o_ref):
    h = jnp.dot(x_ref[:], w_ref[:], preferred_element_type=jnp.float32)
    o_ref[0:N_NODE, :] = h
    for j in range(N_BATCH):
        r = (sidx_ref[j] - 1) % N_NODE
        xr = x_ref[pl.ds(r, 1), :] + pooled_ref[pl.ds(j, 1), :]
        o_ref[pl.ds(r, 1), :] = jnp.dot(xr, w_ref[:],
                                        preferred_element_type=jnp.float32)
    o_ref[0:N_NODE, :] = o_ref[0:N_NODE, :] * _dinv(degp_ref, 1)
    o_ref[N_NODE:N_NODE + 8, :] = jnp.zeros((8, D_FEAT), jnp.float32)


def _fin_glob_body(hg_ref, acc_ref, degp_ref, b_ref, wfc_ref, bfc_ref, o_ref):
    dinv = _dinv(degp_ref, 1)
    pre = (hg_ref[0:N_NODE, :] + acc_ref[0, 0:N_NODE, :]
           + acc_ref[1, 0:N_NODE, :]) * dinv + b_ref[:]
    hg = jnp.maximum(pre, 0.0)
    emb = jnp.sum(hg, axis=0, keepdims=True) / jnp.float32(N_NODE)
    o_ref[:] = jnp.dot(emb, wfc_ref[:],
                       preferred_element_type=jnp.float32) + bfc_ref[:]


def _tc_call(body, out_shape, n_in, smem_args=()):
    in_specs = [pl.BlockSpec(memory_space=pltpu.VMEM) for _ in range(n_in)]
    for i in smem_args:
        in_specs[i] = pl.BlockSpec(memory_space=pltpu.SMEM)
    return pl.pallas_call(body, out_shape=out_shape, in_specs=in_specs)


def _round_up(x, m):
    return ((x + m - 1) // m) * m


@jax.jit
def kernel(sub_x, sub_edge_index, sub_batch, sub_index, global_x,
           global_edge_index, global_batch, W_sub, b_sub, W_glob, b_glob,
           W_fc, b_fc):
    e_sub = sub_edge_index.shape[1]
    e_glob = global_edge_index.shape[1]
    ep_sub = _round_up(e_sub, N_WORKERS * CHUNK * GRP)
    ep_glob = _round_up(e_glob, N_WORKERS * CHUNK * GRP)
    cpt_sub = ep_sub // (N_WORKERS * CHUNK)
    cpt_glob = ep_glob // (N_WORKERS * CHUNK)

    eidx_s = _pack_edges(sub_edge_index, ep_sub)
    eidx_g = _pack_edges(global_edge_index, ep_glob)
    zeros_blk = jnp.zeros((CHUNK, D_FEAT), jnp.float32)

    degp = _make_deg_kernel(cpt_sub, cpt_glob)(eidx_s, eidx_g)
    degp = degp.reshape(2, 2, N_ACC, 1)

    hsub = _tc_call(_mm_sub_body,
                    jax.ShapeDtypeStruct((N_NODE + 8, D_FEAT), jnp.float32),
                    3)(sub_x, W_sub, degp)

    acc_s = _make_rows_kernel(cpt_sub)(hsub, eidx_s, zeros_blk)

    pooled = _tc_call(_fin_sub_body,
                      jax.ShapeDtypeStruct((N_BATCH, D_FEAT), jnp.float32),
                      5)(hsub, acc_s, degp, b_sub.reshape(1, D_FEAT),
                         sub_batch.reshape(N_NODE, 1))

    hglob = _tc_call(_mm_glob_body,
                     jax.ShapeDtypeStruct((N_NODE + 8, D_FEAT), jnp.float32),
                     5, smem_args=(4,))(global_x, W_glob, degp, pooled,
                                        sub_index)

    acc_g = _make_rows_kernel(cpt_glob)(hglob, eidx_g, zeros_blk)

    out = _tc_call(_fin_glob_body,
                   jax.ShapeDtypeStruct((1, D_FEAT), jnp.float32),
                   6)(hglob, acc_g, degp, b_glob.reshape(1, D_FEAT),
                      W_fc, b_fc.reshape(1, D_FEAT))
    return out


# speedup vs baseline: 1.0059x; 1.0059x over previous
import functools

import jax
import jax.numpy as jnp
from jax import lax
from jax.experimental import pallas as pl
from jax.experimental.pallas import tpu as pltpu
from jax.experimental.pallas import tpu_sc as plsc

N_NODE = 10000
D_FEAT = 128
N_ACC = 10112
ROWS_PER_TILE = N_ACC // 16
PAD_IDX = 10000
N_BATCH = 16
CHUNK = 128
N_WORKERS = 32
GRP = 8


def _pack_edges(edge_index, e_pad):
    e = edge_index.shape[1]
    cpt = e_pad // (N_WORKERS * CHUNK)
    padv = jnp.full((e_pad - e,), PAD_IDX, jnp.int32)
    src = jnp.concatenate([edge_index[0], padv]).reshape(N_WORKERS, cpt, CHUNK)
    dst = jnp.concatenate([edge_index[1], padv]).reshape(N_WORKERS, cpt, CHUNK)
    return jnp.concatenate([src, dst], axis=1)


def _sc_mesh():
    return plsc.VectorSubcoreMesh(core_axis_name="c", subcore_axis_name="s")


def _deg_body(cpt_sub, cpt_glob, eidx_sub, eidx_glob, out_hbm,
              idxd_s, idxd_g, ones_v, zbuf_v, deg_sub_sh, deg_glob_sh, sem):
    c = lax.axis_index("c")
    s = lax.axis_index("s")
    wid = c * 16 + s

    pltpu.sync_copy(eidx_sub.at[wid, pl.ds(cpt_sub, cpt_sub)], idxd_s)
    pltpu.sync_copy(eidx_glob.at[wid, pl.ds(cpt_glob, cpt_glob)], idxd_g)

    def fill(i, _):
        ones_v[pl.ds(i * 16, 16)] = jnp.ones((16,), jnp.float32)
        return 0
    lax.fori_loop(0, CHUNK // 16, fill, 0)

    def zfill(i, _):
        zbuf_v[pl.ds(i * 16, 16)] = jnp.zeros((16,), jnp.float32)
        return 0
    lax.fori_loop(0, N_ACC // 16, zfill, 0)

    @pl.when(s == 0)
    def _():
        pltpu.sync_copy(zbuf_v, deg_sub_sh)
        pltpu.sync_copy(zbuf_v, deg_glob_sh)
    plsc.subcore_barrier()

    def scatter_graph(idxd, deg_sh, cpt):
        def body(i, _):
            for j in range(8):
                pltpu.async_copy(ones_v, deg_sh.at[idxd.at[i * 8 + j]], sem,
                                 add=True)
            for j in range(8):
                pltpu.make_async_copy(ones_v, deg_sh.at[idxd.at[0]],
                                      sem).wait()
            return 0
        lax.fori_loop(0, cpt // 8, body, 0)

    scatter_graph(idxd_s, deg_sub_sh, cpt_sub)
    scatter_graph(idxd_g, deg_glob_sh, cpt_glob)
    plsc.subcore_barrier()

    @pl.when(jnp.logical_and(s == 0, c == 0))
    def _():
        pltpu.sync_copy(deg_sub_sh, out_hbm.at[0, 0])
        pltpu.sync_copy(deg_glob_sh, out_hbm.at[1, 0])

    @pl.when(jnp.logical_and(s == 0, c == 1))
    def _():
        pltpu.sync_copy(deg_sub_sh, out_hbm.at[0, 1])
        pltpu.sync_copy(deg_glob_sh, out_hbm.at[1, 1])


def _make_deg_kernel(cpt_sub, cpt_glob):
    return pl.kernel(
        functools.partial(_deg_body, cpt_sub, cpt_glob),
        out_type=jax.ShapeDtypeStruct((2, 2, N_ACC), jnp.float32),
        mesh=_sc_mesh(),
        scratch_types=[
            pltpu.VMEM((cpt_sub, CHUNK), jnp.int32),
            pltpu.VMEM((cpt_glob, CHUNK), jnp.int32),
            pltpu.VMEM((CHUNK,), jnp.float32),
            pltpu.VMEM((N_ACC,), jnp.float32),
            pltpu.VMEM_SHARED((N_ACC,), jnp.float32),
            pltpu.VMEM_SHARED((N_ACC,), jnp.float32),
            pltpu.SemaphoreType.DMA,
        ],
    )


def _rows_body(cpt, h_hbm, src_hbm, dst_hbm, zeros_hbm, out_hbm,
               idx_s, idx_d, rows_v, acc_sh, gsem, ssem):
    c = lax.axis_index("c")
    s = lax.axis_index("s")
    wid = c * 16 + s

    pltpu.sync_copy(zeros_hbm, rows_v.at[0])
    base = s * ROWS_PER_TILE
    for j in range(4):
        pltpu.sync_copy(rows_v.at[0], acc_sh.at[pl.ds(base + j * CHUNK, CHUNK)])
    pltpu.sync_copy(rows_v.at[0, pl.ds(0, ROWS_PER_TILE - 4 * CHUNK)],
                    acc_sh.at[pl.ds(base + 4 * CHUNK, ROWS_PER_TILE - 4 * CHUNK)])
    plsc.subcore_barrier()

    def body(k, _):
        off = (wid * cpt + k) * CHUNK
        pltpu.sync_copy(src_hbm.at[pl.ds(off, CHUNK)], idx_s)
        pltpu.sync_copy(dst_hbm.at[pl.ds(off, CHUNK)], idx_d)
        pltpu.async_copy(h_hbm.at[idx_s], rows_v.at[0], gsem.at[0]).wait()
        pltpu.sync_copy(rows_v.at[0], acc_sh.at[idx_d], add=True)
        return 0
    lax.fori_loop(0, cpt, body, 0)
    plsc.subcore_barrier()

    sizes = [CHUNK] * 4 + [ROWS_PER_TILE - 4 * CHUNK]

    @pl.when(c == 0)
    def _():
        o = 0
        for sz in sizes:
            pltpu.sync_copy(acc_sh.at[pl.ds(base + o, sz)],
                            out_hbm.at[0, pl.ds(base + o, sz)])
            o += sz

    @pl.when(c == 1)
    def _():
        o = 0
        for sz in sizes:
            pltpu.sync_copy(acc_sh.at[pl.ds(base + o, sz)],
                            out_hbm.at[1, pl.ds(base + o, sz)])
            o += sz


def _make_rows_kernel(cpt):
    return pl.kernel(
        functools.partial(_rows_body, cpt),
        out_type=jax.ShapeDtypeStruct((2, N_ACC, D_FEAT), jnp.float32),
        mesh=_sc_mesh(),
        scratch_types=[
            pltpu.VMEM((CHUNK,), jnp.int32),
            pltpu.VMEM((CHUNK,), jnp.int32),
            pltpu.VMEM((2, CHUNK, D_FEAT), jnp.float32),
            pltpu.VMEM_SHARED((N_ACC, D_FEAT), jnp.float32),
            pltpu.SemaphoreType.DMA((2,)),
            pltpu.SemaphoreType.DMA((2,)),
        ],
    )


def _dinv(degp_ref, g):
    deg = degp_ref[g, 0, 0:N_NODE, :] + degp_ref[g, 1, 0:N_NODE, :] + 1.0
    return lax.rsqrt(jnp.maximum(deg, 1e-12))


def _mm_sub_body(x_ref, w_ref, degp_ref, o_ref):
    h = jnp.dot(x_ref[:], w_ref[:], preferred_element_type=jnp.float32)
    o_ref[0:N_NODE, :] = h * _dinv(degp_ref, 0)
    o_ref[N_NODE:N_NODE + 8, :] = jnp.zeros((8, D_FEAT), jnp.float32)


def _fin_sub_body(hsub_ref, acc_ref, degp_ref, b_ref, batch_ref, o_ref):
    dinv = _dinv(degp_ref, 0)
    pre = (hsub_ref[0:N_NODE, :] + acc_ref[0, 0:N_NODE, :]
           + acc_ref[1, 0:N_NODE, :]) * dinv + b_ref[:]
    hs = jnp.maximum(pre, 0.0)
    onehot = (batch_ref[:] == lax.broadcasted_iota(
        jnp.int32, (N_NODE, N_BATCH), 1)).astype(jnp.float32)
    dn = (((0,), (0,)), ((), ()))
    psum = lax.dot_general(onehot, hs, dn,
                           preferred_element_type=jnp.float32)
    cnt = lax.dot_general(onehot, jnp.ones((N_NODE, 1), jnp.float32), dn,
                          preferred_element_type=jnp.float32)
    o_ref[:] = psum / jnp.maximum(cnt, 1.0)


def _mm_glob_body(x_ref, w_ref, degp_ref, pooled_ref, sidx_ref, o_ref):
    h = jnp.dot(x_ref[:], w_ref[:], preferred_element_type=jnp.float32)
    o_ref[0:N_NODE, :] = h
    for j in range(N_BATCH):
        r = (sidx_ref[j] - 1) % N_NODE
        xr = x_ref[pl.ds(r, 1), :] + pooled_ref[pl.ds(j, 1), :]
        o_ref[pl.ds(r, 1), :] = jnp.dot(xr, w_ref[:],
                                        preferred_element_type=jnp.float32)
    o_ref[0:N_NODE, :] = o_ref[0:N_NODE, :] * _dinv(degp_ref, 1)
    o_ref[N_NODE:N_NODE + 8, :] = jnp.zeros((8, D_FEAT), jnp.float32)


def _fin_glob_body(hg_ref, acc_ref, degp_ref, b_ref, wfc_ref, bfc_ref, o_ref):
    dinv = _dinv(degp_ref, 1)
    pre = (hg_ref[0:N_NODE, :] + acc_ref[0, 0:N_NODE, :]
           + acc_ref[1, 0:N_NODE, :]) * dinv + b_ref[:]
    hg = jnp.maximum(pre, 0.0)
    emb = jnp.sum(hg, axis=0, keepdims=True) / jnp.float32(N_NODE)
    o_ref[:] = jnp.dot(emb, wfc_ref[:],
                       preferred_element_type=jnp.float32) + bfc_ref[:]


def _tc_call(body, out_shape, n_in, smem_args=()):
    in_specs = [pl.BlockSpec(memory_space=pltpu.VMEM) for _ in range(n_in)]
    for i in smem_args:
        in_specs[i] = pl.BlockSpec(memory_space=pltpu.SMEM)
    return pl.pallas_call(body, out_shape=out_shape, in_specs=in_specs)


def _round_up(x, m):
    return ((x + m - 1) // m) * m


@jax.jit
def kernel(sub_x, sub_edge_index, sub_batch, sub_index, global_x,
           global_edge_index, global_batch, W_sub, b_sub, W_glob, b_glob,
           W_fc, b_fc):
    e_sub = sub_edge_index.shape[1]
    e_glob = global_edge_index.shape[1]
    ep_sub = _round_up(e_sub, N_WORKERS * CHUNK * GRP)
    ep_glob = _round_up(e_glob, N_WORKERS * CHUNK * GRP)
    cpt_sub = ep_sub // (N_WORKERS * CHUNK)
    cpt_glob = ep_glob // (N_WORKERS * CHUNK)

    eidx_s = _pack_edges(sub_edge_index, ep_sub)
    eidx_g = _pack_edges(global_edge_index, ep_glob)
    src_s, dst_s = eidx_s[:, :cpt_sub].reshape(-1), eidx_s[:, cpt_sub:].reshape(-1)
    src_g, dst_g = eidx_g[:, :cpt_glob].reshape(-1), eidx_g[:, cpt_glob:].reshape(-1)
    zeros_blk = jnp.zeros((CHUNK, D_FEAT), jnp.float32)

    degp = _make_deg_kernel(cpt_sub, cpt_glob)(eidx_s, eidx_g)
    degp = degp.reshape(2, 2, N_ACC, 1)

    hsub = _tc_call(_mm_sub_body,
                    jax.ShapeDtypeStruct((N_NODE + 8, D_FEAT), jnp.float32),
                    3)(sub_x, W_sub, degp)

    acc_s = _make_rows_kernel(cpt_sub)(hsub, src_s, dst_s, zeros_blk)

    pooled = _tc_call(_fin_sub_body,
                      jax.ShapeDtypeStruct((N_BATCH, D_FEAT), jnp.float32),
                      5)(hsub, acc_s, degp, b_sub.reshape(1, D_FEAT),
                         sub_batch.reshape(N_NODE, 1))

    hglob = _tc_call(_mm_glob_body,
                     jax.ShapeDtypeStruct((N_NODE + 8, D_FEAT), jnp.float32),
                     5, smem_args=(4,))(global_x, W_glob, degp, pooled,
                                        sub_index)

    acc_g = _make_rows_kernel(cpt_glob)(hglob, src_g, dst_g, zeros_blk)

    out = _tc_call(_fin_glob_body,
                   jax.ShapeDtypeStruct((1, D_FEAT), jnp.float32),
                   6)(hglob, acc_g, degp, b_glob.reshape(1, D_FEAT),
                      W_fc, b_fc.reshape(1, D_FEAT))
    return out


# speedup vs baseline: 2.0041x; 1.9923x over previous
import functools

import jax
import jax.numpy as jnp
from jax import lax
from jax.experimental import pallas as pl
from jax.experimental.pallas import tpu as pltpu
from jax.experimental.pallas import tpu_sc as plsc

N_NODE = 10000
D_FEAT = 128
N_ACC = 10112
ROWS_PER_TILE = N_ACC // 16
PAD_IDX = 10000
N_BATCH = 16
CHUNK = 128
N_WORKERS = 32
GRP = 8


def _pack_edges(edge_index, e_pad):
    e = edge_index.shape[1]
    cpt = e_pad // (N_WORKERS * CHUNK)
    padv = PAD_IDX + jnp.arange(e_pad - e, dtype=jnp.int32) % (N_ACC - PAD_IDX)
    src = jnp.concatenate([edge_index[0], padv]).reshape(N_WORKERS, cpt, CHUNK)
    dst = jnp.concatenate([edge_index[1], padv]).reshape(N_WORKERS, cpt, CHUNK)
    return jnp.concatenate([src, dst], axis=1)


def _sc_mesh():
    return plsc.VectorSubcoreMesh(core_axis_name="c", subcore_axis_name="s")


def _deg_body(cpt_sub, cpt_glob, eidx_sub, eidx_glob, out_hbm,
              idxd_s, idxd_g, ones_v, zbuf_v, deg_sub_sh, deg_glob_sh, sem):
    c = lax.axis_index("c")
    s = lax.axis_index("s")
    wid = c * 16 + s

    pltpu.sync_copy(eidx_sub.at[wid, pl.ds(cpt_sub, cpt_sub)], idxd_s)
    pltpu.sync_copy(eidx_glob.at[wid, pl.ds(cpt_glob, cpt_glob)], idxd_g)

    def fill(i, _):
        ones_v[pl.ds(i * 16, 16)] = jnp.ones((16,), jnp.float32)
        return 0
    lax.fori_loop(0, CHUNK // 16, fill, 0)

    def zfill(i, _):
        zbuf_v[pl.ds(i * 16, 16)] = jnp.zeros((16,), jnp.float32)
        return 0
    lax.fori_loop(0, N_ACC // 16, zfill, 0)

    @pl.when(s == 0)
    def _():
        pltpu.sync_copy(zbuf_v, deg_sub_sh)
        pltpu.sync_copy(zbuf_v, deg_glob_sh)
    plsc.subcore_barrier()

    def scatter_graph(idxd, deg_sh, cpt):
        def body(i, _):
            for j in range(8):
                pltpu.async_copy(ones_v, deg_sh.at[idxd.at[i * 8 + j]], sem,
                                 add=True)
            for j in range(8):
                pltpu.make_async_copy(ones_v, deg_sh.at[idxd.at[0]],
                                      sem).wait()
            return 0
        lax.fori_loop(0, cpt // 8, body, 0)

    scatter_graph(idxd_s, deg_sub_sh, cpt_sub)
    scatter_graph(idxd_g, deg_glob_sh, cpt_glob)
    plsc.subcore_barrier()

    @pl.when(jnp.logical_and(s == 0, c == 0))
    def _():
        pltpu.sync_copy(deg_sub_sh, out_hbm.at[0, 0])
        pltpu.sync_copy(deg_glob_sh, out_hbm.at[1, 0])

    @pl.when(jnp.logical_and(s == 0, c == 1))
    def _():
        pltpu.sync_copy(deg_sub_sh, out_hbm.at[0, 1])
        pltpu.sync_copy(deg_glob_sh, out_hbm.at[1, 1])


def _make_deg_kernel(cpt_sub, cpt_glob):
    return pl.kernel(
        functools.partial(_deg_body, cpt_sub, cpt_glob),
        out_type=jax.ShapeDtypeStruct((2, 2, N_ACC), jnp.float32),
        mesh=_sc_mesh(),
        scratch_types=[
            pltpu.VMEM((cpt_sub, CHUNK), jnp.int32),
            pltpu.VMEM((cpt_glob, CHUNK), jnp.int32),
            pltpu.VMEM((CHUNK,), jnp.float32),
            pltpu.VMEM((N_ACC,), jnp.float32),
            pltpu.VMEM_SHARED((N_ACC,), jnp.float32),
            pltpu.VMEM_SHARED((N_ACC,), jnp.float32),
            pltpu.SemaphoreType.DMA,
        ],
    )


def _rows_body(cpt, h_hbm, src_hbm, dst_hbm, zeros_hbm, out_hbm,
               idx_s, idx_d, rows_v, acc_sh, gsem, ssem):
    c = lax.axis_index("c")
    s = lax.axis_index("s")
    wid = c * 16 + s

    pltpu.sync_copy(zeros_hbm, rows_v.at[0])
    base = s * ROWS_PER_TILE
    for j in range(4):
        pltpu.sync_copy(rows_v.at[0], acc_sh.at[pl.ds(base + j * CHUNK, CHUNK)])
    pltpu.sync_copy(rows_v.at[0, pl.ds(0, ROWS_PER_TILE - 4 * CHUNK)],
                    acc_sh.at[pl.ds(base + 4 * CHUNK, ROWS_PER_TILE - 4 * CHUNK)])
    plsc.subcore_barrier()

    def body(k, _):
        off = (wid * cpt + k) * CHUNK
        pltpu.sync_copy(src_hbm.at[pl.ds(off, CHUNK)], idx_s)
        pltpu.sync_copy(dst_hbm.at[pl.ds(off, CHUNK)], idx_d)
        pltpu.async_copy(h_hbm.at[idx_s], rows_v.at[0], gsem.at[0]).wait()
        pltpu.sync_copy(rows_v.at[0], acc_sh.at[idx_d], add=True)
        return 0
    lax.fori_loop(0, cpt, body, 0)
    plsc.subcore_barrier()

    sizes = [CHUNK] * 4 + [ROWS_PER_TILE - 4 * CHUNK]

    @pl.when(c == 0)
    def _():
        o = 0
        for sz in sizes:
            pltpu.sync_copy(acc_sh.at[pl.ds(base + o, sz)],
                            out_hbm.at[0, pl.ds(base + o, sz)])
            o += sz

    @pl.when(c == 1)
    def _():
        o = 0
        for sz in sizes:
            pltpu.sync_copy(acc_sh.at[pl.ds(base + o, sz)],
                            out_hbm.at[1, pl.ds(base + o, sz)])
            o += sz


def _make_rows_kernel(cpt):
    return pl.kernel(
        functools.partial(_rows_body, cpt),
        out_type=jax.ShapeDtypeStruct((2, N_ACC, D_FEAT), jnp.float32),
        mesh=_sc_mesh(),
        scratch_types=[
            pltpu.VMEM((CHUNK,), jnp.int32),
            pltpu.VMEM((CHUNK,), jnp.int32),
            pltpu.VMEM((2, CHUNK, D_FEAT), jnp.float32),
            pltpu.VMEM_SHARED((N_ACC, D_FEAT), jnp.float32),
            pltpu.SemaphoreType.DMA((2,)),
            pltpu.SemaphoreType.DMA((2,)),
        ],
    )


def _dinv(degp_ref, g):
    deg = degp_ref[g, 0, 0:N_NODE, :] + degp_ref[g, 1, 0:N_NODE, :] + 1.0
    return lax.rsqrt(jnp.maximum(deg, 1e-12))


def _mm_sub_body(x_ref, w_ref, degp_ref, o_ref):
    h = jnp.dot(x_ref[:], w_ref[:], preferred_element_type=jnp.float32)
    o_ref[0:N_NODE, :] = h * _dinv(degp_ref, 0)
    o_ref[N_NODE:N_ACC, :] = jnp.zeros((N_ACC - N_NODE, D_FEAT), jnp.float32)


def _fin_sub_body(hsub_ref, acc_ref, degp_ref, b_ref, batch_ref, o_ref):
    dinv = _dinv(degp_ref, 0)
    pre = (hsub_ref[0:N_NODE, :] + acc_ref[0, 0:N_NODE, :]
           + acc_ref[1, 0:N_NODE, :]) * dinv + b_ref[:]
    hs = jnp.maximum(pre, 0.0)
    onehot = (batch_ref[:] == lax.broadcasted_iota(
        jnp.int32, (N_NODE, N_BATCH), 1)).astype(jnp.float32)
    dn = (((0,), (0,)), ((), ()))
    psum = lax.dot_general(onehot, hs, dn,
                           preferred_element_type=jnp.float32)
    cnt = lax.dot_general(onehot, jnp.ones((N_NODE, 1), jnp.float32), dn,
                          preferred_element_type=jnp.float32)
    o_ref[:] = psum / jnp.maximum(cnt, 1.0)


def _mm_glob_body(x_ref, w_ref, degp_ref, pooled_ref, sidx_ref, o_ref):
    h = jnp.dot(x_ref[:], w_ref[:], preferred_element_type=jnp.float32)
    o_ref[0:N_NODE, :] = h
    for j in range(N_BATCH):
        r = (sidx_ref[j] - 1) % N_NODE
        xr = x_ref[pl.ds(r, 1), :] + pooled_ref[pl.ds(j, 1), :]
        o_ref[pl.ds(r, 1), :] = jnp.dot(xr, w_ref[:],
                                        preferred_element_type=jnp.float32)
    o_ref[0:N_NODE, :] = o_ref[0:N_NODE, :] * _dinv(degp_ref, 1)
    o_ref[N_NODE:N_ACC, :] = jnp.zeros((N_ACC - N_NODE, D_FEAT), jnp.float32)


def _fin_glob_body(hg_ref, acc_ref, degp_ref, b_ref, wfc_ref, bfc_ref, o_ref):
    dinv = _dinv(degp_ref, 1)
    pre = (hg_ref[0:N_NODE, :] + acc_ref[0, 0:N_NODE, :]
           + acc_ref[1, 0:N_NODE, :]) * dinv + b_ref[:]
    hg = jnp.maximum(pre, 0.0)
    emb = jnp.sum(hg, axis=0, keepdims=True) / jnp.float32(N_NODE)
    o_ref[:] = jnp.dot(emb, wfc_ref[:],
                       preferred_element_type=jnp.float32) + bfc_ref[:]


def _tc_call(body, out_shape, n_in, smem_args=()):
    in_specs = [pl.BlockSpec(memory_space=pltpu.VMEM) for _ in range(n_in)]
    for i in smem_args:
        in_specs[i] = pl.BlockSpec(memory_space=pltpu.SMEM)
    return pl.pallas_call(body, out_shape=out_shape, in_specs=in_specs)


def _round_up(x, m):
    return ((x + m - 1) // m) * m


@jax.jit
def kernel(sub_x, sub_edge_index, sub_batch, sub_index, global_x,
           global_edge_index, global_batch, W_sub, b_sub, W_glob, b_glob,
           W_fc, b_fc):
    e_sub = sub_edge_index.shape[1]
    e_glob = global_edge_index.shape[1]
    ep_sub = _round_up(e_sub, N_WORKERS * CHUNK * GRP)
    ep_glob = _round_up(e_glob, N_WORKERS * CHUNK * GRP)
    cpt_sub = ep_sub // (N_WORKERS * CHUNK)
    cpt_glob = ep_glob // (N_WORKERS * CHUNK)

    eidx_s = _pack_edges(sub_edge_index, ep_sub)
    eidx_g = _pack_edges(global_edge_index, ep_glob)
    src_s, dst_s = eidx_s[:, :cpt_sub].reshape(-1), eidx_s[:, cpt_sub:].reshape(-1)
    src_g, dst_g = eidx_g[:, :cpt_glob].reshape(-1), eidx_g[:, cpt_glob:].reshape(-1)
    zeros_blk = jnp.zeros((CHUNK, D_FEAT), jnp.float32)

    degp = _make_deg_kernel(cpt_sub, cpt_glob)(eidx_s, eidx_g)
    degp = degp.reshape(2, 2, N_ACC, 1)

    hsub = _tc_call(_mm_sub_body,
                    jax.ShapeDtypeStruct((N_ACC, D_FEAT), jnp.float32),
                    3)(sub_x, W_sub, degp)

    acc_s = _make_rows_kernel(cpt_sub)(hsub, src_s, dst_s, zeros_blk)

    pooled = _tc_call(_fin_sub_body,
                      jax.ShapeDtypeStruct((N_BATCH, D_FEAT), jnp.float32),
                      5)(hsub, acc_s, degp, b_sub.reshape(1, D_FEAT),
                         sub_batch.reshape(N_NODE, 1))

    hglob = _tc_call(_mm_glob_body,
                     jax.ShapeDtypeStruct((N_ACC, D_FEAT), jnp.float32),
                     5, smem_args=(4,))(global_x, W_glob, degp, pooled,
                                        sub_index)

    acc_g = _make_rows_kernel(cpt_glob)(hglob, src_g, dst_g, zeros_blk)

    out = _tc_call(_fin_glob_body,
                   jax.ShapeDtypeStruct((1, D_FEAT), jnp.float32),
                   6)(hglob, acc_g, degp, b_glob.reshape(1, D_FEAT),
                      W_fc, b_fc.reshape(1, D_FEAT))
    return out


# speedup vs baseline: 2.3530x; 1.1741x over previous
import functools

import jax
import jax.numpy as jnp
from jax import lax
from jax.experimental import pallas as pl
from jax.experimental.pallas import tpu as pltpu
from jax.experimental.pallas import tpu_sc as plsc

N_NODE = 10000
D_FEAT = 128
N_ACC = 10112
ROWS_PER_TILE = N_ACC // 16
PAD_IDX = 10000
N_BATCH = 16
CHUNK = 128
N_WORKERS = 32
GRP = 8


def _pack_edges(edge_index, e_pad):
    e = edge_index.shape[1]
    cpt = e_pad // (N_WORKERS * CHUNK)
    padv = PAD_IDX + jnp.arange(e_pad - e, dtype=jnp.int32) % (N_ACC - PAD_IDX)
    src = jnp.concatenate([edge_index[0], padv]).reshape(N_WORKERS, cpt, CHUNK)
    dst = jnp.concatenate([edge_index[1], padv]).reshape(N_WORKERS, cpt, CHUNK)
    return jnp.concatenate([src, dst], axis=1)


def _sc_mesh():
    return plsc.VectorSubcoreMesh(core_axis_name="c", subcore_axis_name="s")


def _deg_body(cpt_sub, cpt_glob, eidx_sub, eidx_glob, out_hbm,
              idxd_s, idxd_g, ones_v, zbuf_v, deg_sub_sh, deg_glob_sh, sem):
    c = lax.axis_index("c")
    s = lax.axis_index("s")
    wid = c * 16 + s

    pltpu.sync_copy(eidx_sub.at[wid, pl.ds(cpt_sub, cpt_sub)], idxd_s)
    pltpu.sync_copy(eidx_glob.at[wid, pl.ds(cpt_glob, cpt_glob)], idxd_g)

    def fill(i, _):
        ones_v[pl.ds(i * 16, 16)] = jnp.ones((16,), jnp.float32)
        return 0
    lax.fori_loop(0, CHUNK // 16, fill, 0)

    def zfill(i, _):
        zbuf_v[pl.ds(i * 16, 16)] = jnp.zeros((16,), jnp.float32)
        return 0
    lax.fori_loop(0, N_ACC // 16, zfill, 0)

    @pl.when(s == 0)
    def _():
        pltpu.sync_copy(zbuf_v, deg_sub_sh)
        pltpu.sync_copy(zbuf_v, deg_glob_sh)
    plsc.subcore_barrier()

    def scatter_graph(idxd, deg_sh, cpt):
        def body(i, _):
            for j in range(8):
                pltpu.async_copy(ones_v, deg_sh.at[idxd.at[i * 8 + j]], sem,
                                 add=True)
            for j in range(8):
                pltpu.make_async_copy(ones_v, deg_sh.at[idxd.at[0]],
                                      sem).wait()
            return 0
        lax.fori_loop(0, cpt // 8, body, 0)

    scatter_graph(idxd_s, deg_sub_sh, cpt_sub)
    scatter_graph(idxd_g, deg_glob_sh, cpt_glob)
    plsc.subcore_barrier()

    @pl.when(jnp.logical_and(s == 0, c == 0))
    def _():
        pltpu.sync_copy(deg_sub_sh, out_hbm.at[0, 0])
        pltpu.sync_copy(deg_glob_sh, out_hbm.at[1, 0])

    @pl.when(jnp.logical_and(s == 0, c == 1))
    def _():
        pltpu.sync_copy(deg_sub_sh, out_hbm.at[0, 1])
        pltpu.sync_copy(deg_glob_sh, out_hbm.at[1, 1])


def _make_deg_kernel(cpt_sub, cpt_glob):
    return pl.kernel(
        functools.partial(_deg_body, cpt_sub, cpt_glob),
        out_type=jax.ShapeDtypeStruct((2, 2, N_ACC), jnp.float32),
        mesh=_sc_mesh(),
        scratch_types=[
            pltpu.VMEM((cpt_sub, CHUNK), jnp.int32),
            pltpu.VMEM((cpt_glob, CHUNK), jnp.int32),
            pltpu.VMEM((CHUNK,), jnp.float32),
            pltpu.VMEM((N_ACC,), jnp.float32),
            pltpu.VMEM_SHARED((N_ACC,), jnp.float32),
            pltpu.VMEM_SHARED((N_ACC,), jnp.float32),
            pltpu.SemaphoreType.DMA,
        ],
    )


def _rows_body(cpt, h_hbm, src_hbm, dst_hbm, zeros_hbm, out_hbm,
               idx_s, idx_d, rows_v, acc_sh, gsem, ssem):
    c = lax.axis_index("c")
    s = lax.axis_index("s")
    wid = c * 16 + s

    pltpu.sync_copy(zeros_hbm, rows_v.at[0])
    base = s * ROWS_PER_TILE
    for j in range(4):
        pltpu.sync_copy(rows_v.at[0], acc_sh.at[pl.ds(base + j * CHUNK, CHUNK)])
    pltpu.sync_copy(rows_v.at[0, pl.ds(0, ROWS_PER_TILE - 4 * CHUNK)],
                    acc_sh.at[pl.ds(base + 4 * CHUNK, ROWS_PER_TILE - 4 * CHUNK)])
    plsc.subcore_barrier()

    def body(i, _):
        for b in range(2):
            k = 2 * i + b
            off = (wid * cpt + k) * CHUNK
            pltpu.sync_copy(src_hbm.at[pl.ds(off, CHUNK)], idx_s)
            pltpu.sync_copy(dst_hbm.at[pl.ds(off, CHUNK)], idx_d.at[b])
            pltpu.async_copy(h_hbm.at[idx_s], rows_v.at[b], gsem.at[b])

            @pl.when(k >= 1)
            def _():
                pltpu.make_async_copy(rows_v.at[1 - b],
                                      acc_sh.at[idx_d.at[1 - b]],
                                      ssem.at[1 - b]).wait()
            pltpu.make_async_copy(h_hbm.at[idx_s], rows_v.at[b],
                                  gsem.at[b]).wait()
            pltpu.async_copy(rows_v.at[b], acc_sh.at[idx_d.at[b]],
                             ssem.at[b], add=True)
        return 0
    lax.fori_loop(0, cpt // 2, body, 0)
    pltpu.make_async_copy(rows_v.at[1], acc_sh.at[idx_d.at[1]],
                          ssem.at[1]).wait()
    plsc.subcore_barrier()

    sizes = [CHUNK] * 4 + [ROWS_PER_TILE - 4 * CHUNK]

    @pl.when(c == 0)
    def _():
        o = 0
        for sz in sizes:
            pltpu.sync_copy(acc_sh.at[pl.ds(base + o, sz)],
                            out_hbm.at[0, pl.ds(base + o, sz)])
            o += sz

    @pl.when(c == 1)
    def _():
        o = 0
        for sz in sizes:
            pltpu.sync_copy(acc_sh.at[pl.ds(base + o, sz)],
                            out_hbm.at[1, pl.ds(base + o, sz)])
            o += sz


def _make_rows_kernel(cpt):
    return pl.kernel(
        functools.partial(_rows_body, cpt),
        out_type=jax.ShapeDtypeStruct((2, N_ACC, D_FEAT), jnp.float32),
        mesh=_sc_mesh(),
        scratch_types=[
            pltpu.VMEM((CHUNK,), jnp.int32),
            pltpu.VMEM((2, CHUNK), jnp.int32),
            pltpu.VMEM((2, CHUNK, D_FEAT), jnp.float32),
            pltpu.VMEM_SHARED((N_ACC, D_FEAT), jnp.float32),
            pltpu.SemaphoreType.DMA((2,)),
            pltpu.SemaphoreType.DMA((2,)),
        ],
    )


def _dinv(degp_ref, g):
    deg = degp_ref[g, 0, 0:N_NODE, :] + degp_ref[g, 1, 0:N_NODE, :] + 1.0
    return lax.rsqrt(jnp.maximum(deg, 1e-12))


def _mm_sub_body(x_ref, w_ref, degp_ref, o_ref):
    h = jnp.dot(x_ref[:], w_ref[:], preferred_element_type=jnp.float32)
    o_ref[0:N_NODE, :] = h * _dinv(degp_ref, 0)
    o_ref[N_NODE:N_ACC, :] = jnp.zeros((N_ACC - N_NODE, D_FEAT), jnp.float32)


def _fin_sub_body(hsub_ref, acc_ref, degp_ref, b_ref, batch_ref, o_ref):
    dinv = _dinv(degp_ref, 0)
    pre = (hsub_ref[0:N_NODE, :] + acc_ref[0, 0:N_NODE, :]
           + acc_ref[1, 0:N_NODE, :]) * dinv + b_ref[:]
    hs = jnp.maximum(pre, 0.0)
    onehot = (batch_ref[:] == lax.broadcasted_iota(
        jnp.int32, (N_NODE, N_BATCH), 1)).astype(jnp.float32)
    dn = (((0,), (0,)), ((), ()))
    psum = lax.dot_general(onehot, hs, dn,
                           preferred_element_type=jnp.float32)
    cnt = lax.dot_general(onehot, jnp.ones((N_NODE, 1), jnp.float32), dn,
                          preferred_element_type=jnp.float32)
    o_ref[:] = psum / jnp.maximum(cnt, 1.0)


def _mm_glob_body(x_ref, w_ref, degp_ref, pooled_ref, sidx_ref, o_ref):
    h = jnp.dot(x_ref[:], w_ref[:], preferred_element_type=jnp.float32)
    o_ref[0:N_NODE, :] = h
    for j in range(N_BATCH):
        r = (sidx_ref[j] - 1) % N_NODE
        xr = x_ref[pl.ds(r, 1), :] + pooled_ref[pl.ds(j, 1), :]
        o_ref[pl.ds(r, 1), :] = jnp.dot(xr, w_ref[:],
                                        preferred_element_type=jnp.float32)
    o_ref[0:N_NODE, :] = o_ref[0:N_NODE, :] * _dinv(degp_ref, 1)
    o_ref[N_NODE:N_ACC, :] = jnp.zeros((N_ACC - N_NODE, D_FEAT), jnp.float32)


def _fin_glob_body(hg_ref, acc_ref, degp_ref, b_ref, wfc_ref, bfc_ref, o_ref):
    dinv = _dinv(degp_ref, 1)
    pre = (hg_ref[0:N_NODE, :] + acc_ref[0, 0:N_NODE, :]
           + acc_ref[1, 0:N_NODE, :]) * dinv + b_ref[:]
    hg = jnp.maximum(pre, 0.0)
    emb = jnp.sum(hg, axis=0, keepdims=True) / jnp.float32(N_NODE)
    o_ref[:] = jnp.dot(emb, wfc_ref[:],
                       preferred_element_type=jnp.float32) + bfc_ref[:]


def _tc_call(body, out_shape, n_in, smem_args=()):
    in_specs = [pl.BlockSpec(memory_space=pltpu.VMEM) for _ in range(n_in)]
    for i in smem_args:
        in_specs[i] = pl.BlockSpec(memory_space=pltpu.SMEM)
    return pl.pallas_call(body, out_shape=out_shape, in_specs=in_specs)


def _round_up(x, m):
    return ((x + m - 1) // m) * m


@jax.jit
def kernel(sub_x, sub_edge_index, sub_batch, sub_index, global_x,
           global_edge_index, global_batch, W_sub, b_sub, W_glob, b_glob,
           W_fc, b_fc):
    e_sub = sub_edge_index.shape[1]
    e_glob = global_edge_index.shape[1]
    ep_sub = _round_up(e_sub, N_WORKERS * CHUNK * GRP)
    ep_glob = _round_up(e_glob, N_WORKERS * CHUNK * GRP)
    cpt_sub = ep_sub // (N_WORKERS * CHUNK)
    cpt_glob = ep_glob // (N_WORKERS * CHUNK)

    eidx_s = _pack_edges(sub_edge_index, ep_sub)
    eidx_g = _pack_edges(global_edge_index, ep_glob)
    src_s, dst_s = eidx_s[:, :cpt_sub].reshape(-1), eidx_s[:, cpt_sub:].reshape(-1)
    src_g, dst_g = eidx_g[:, :cpt_glob].reshape(-1), eidx_g[:, cpt_glob:].reshape(-1)
    zeros_blk = jnp.zeros((CHUNK, D_FEAT), jnp.float32)

    degp = _make_deg_kernel(cpt_sub, cpt_glob)(eidx_s, eidx_g)
    degp = degp.reshape(2, 2, N_ACC, 1)

    hsub = _tc_call(_mm_sub_body,
                    jax.ShapeDtypeStruct((N_ACC, D_FEAT), jnp.float32),
                    3)(sub_x, W_sub, degp)

    acc_s = _make_rows_kernel(cpt_sub)(hsub, src_s, dst_s, zeros_blk)

    pooled = _tc_call(_fin_sub_body,
                      jax.ShapeDtypeStruct((N_BATCH, D_FEAT), jnp.float32),
                      5)(hsub, acc_s, degp, b_sub.reshape(1, D_FEAT),
                         sub_batch.reshape(N_NODE, 1))

    hglob = _tc_call(_mm_glob_body,
                     jax.ShapeDtypeStruct((N_ACC, D_FEAT), jnp.float32),
                     5, smem_args=(4,))(global_x, W_glob, degp, pooled,
                                        sub_index)

    acc_g = _make_rows_kernel(cpt_glob)(hglob, src_g, dst_g, zeros_blk)

    out = _tc_call(_fin_glob_body,
                   jax.ShapeDtypeStruct((1, D_FEAT), jnp.float32),
                   6)(hglob, acc_g, degp, b_glob.reshape(1, D_FEAT),
                      W_fc, b_fc.reshape(1, D_FEAT))
    return out


# speedup vs baseline: 3.0207x; 1.2838x over previous
import functools

import jax
import jax.numpy as jnp
from jax import lax
from jax.experimental import pallas as pl
from jax.experimental.pallas import tpu as pltpu
from jax.experimental.pallas import tpu_sc as plsc

N_NODE = 10000
D_FEAT = 128
N_ACC = 10112
ROWS_PER_TILE = N_ACC // 16
PAD_IDX = 10000
N_BATCH = 16
CHUNK = 128
N_WORKERS = 32
GRP = 8


def _pack_edges(edge_index, e_pad):
    e = edge_index.shape[1]
    cpt = e_pad // (N_WORKERS * CHUNK)
    padv = PAD_IDX + jnp.arange(e_pad - e, dtype=jnp.int32) % (N_ACC - PAD_IDX)
    src = jnp.concatenate([edge_index[0], padv]).reshape(N_WORKERS, cpt, CHUNK)
    dst = jnp.concatenate([edge_index[1], padv]).reshape(N_WORKERS, cpt, CHUNK)
    return jnp.concatenate([src, dst], axis=1)


def _sc_mesh():
    return plsc.VectorSubcoreMesh(core_axis_name="c", subcore_axis_name="s")


def _deg_body(cpt_sub, cpt_glob, eidx_sub, eidx_glob, out_hbm,
              idxd_s, idxd_g, ones_v, zbuf_v, deg_sub_sh, deg_glob_sh, sem):
    c = lax.axis_index("c")
    s = lax.axis_index("s")
    wid = c * 16 + s

    pltpu.sync_copy(eidx_sub.at[wid, pl.ds(cpt_sub, cpt_sub)], idxd_s)
    pltpu.sync_copy(eidx_glob.at[wid, pl.ds(cpt_glob, cpt_glob)], idxd_g)

    def fill(i, _):
        ones_v[pl.ds(i * 16, 16)] = jnp.ones((16,), jnp.float32)
        return 0
    lax.fori_loop(0, CHUNK // 16, fill, 0)

    def zfill(i, _):
        zbuf_v[pl.ds(i * 16, 16)] = jnp.zeros((16,), jnp.float32)
        return 0
    lax.fori_loop(0, N_ACC // 16, zfill, 0)

    @pl.when(s == 0)
    def _():
        pltpu.sync_copy(zbuf_v, deg_sub_sh)
        pltpu.sync_copy(zbuf_v, deg_glob_sh)
    plsc.subcore_barrier()

    def scatter_graph(idxd, deg_sh, cpt):
        def body(i, _):
            for j in range(8):
                pltpu.async_copy(ones_v, deg_sh.at[idxd.at[i * 8 + j]], sem,
                                 add=True)
            for j in range(8):
                pltpu.make_async_copy(ones_v, deg_sh.at[idxd.at[0]],
                                      sem).wait()
            return 0
        lax.fori_loop(0, cpt // 8, body, 0)

    scatter_graph(idxd_s, deg_sub_sh, cpt_sub)
    scatter_graph(idxd_g, deg_glob_sh, cpt_glob)
    plsc.subcore_barrier()

    @pl.when(jnp.logical_and(s == 0, c == 0))
    def _():
        pltpu.sync_copy(deg_sub_sh, out_hbm.at[0, 0])
        pltpu.sync_copy(deg_glob_sh, out_hbm.at[1, 0])

    @pl.when(jnp.logical_and(s == 0, c == 1))
    def _():
        pltpu.sync_copy(deg_sub_sh, out_hbm.at[0, 1])
        pltpu.sync_copy(deg_glob_sh, out_hbm.at[1, 1])


def _make_deg_kernel(cpt_sub, cpt_glob):
    return pl.kernel(
        functools.partial(_deg_body, cpt_sub, cpt_glob),
        out_type=jax.ShapeDtypeStruct((2, 2, N_ACC), jnp.float32),
        mesh=_sc_mesh(),
        scratch_types=[
            pltpu.VMEM((cpt_sub, CHUNK), jnp.int32),
            pltpu.VMEM((cpt_glob, CHUNK), jnp.int32),
            pltpu.VMEM((CHUNK,), jnp.float32),
            pltpu.VMEM((N_ACC,), jnp.float32),
            pltpu.VMEM_SHARED((N_ACC,), jnp.float32),
            pltpu.VMEM_SHARED((N_ACC,), jnp.float32),
            pltpu.SemaphoreType.DMA,
        ],
    )


def _rows_body(cpt, h_hbm, eidx_hbm, zeros_hbm, out_hbm,
               idxd, sbuf, rows_v, acc_sh, gsem, ssem):
    c = lax.axis_index("c")
    s = lax.axis_index("s")
    wid = c * 16 + s

    pltpu.sync_copy(eidx_hbm.at[wid, pl.ds(cpt, cpt)], idxd)

    pltpu.sync_copy(zeros_hbm, rows_v.at[0])
    base = s * ROWS_PER_TILE
    for j in range(4):
        pltpu.sync_copy(rows_v.at[0], acc_sh.at[pl.ds(base + j * CHUNK, CHUNK)])
    pltpu.sync_copy(rows_v.at[0, pl.ds(0, ROWS_PER_TILE - 4 * CHUNK)],
                    acc_sh.at[pl.ds(base + 4 * CHUNK, ROWS_PER_TILE - 4 * CHUNK)])
    plsc.subcore_barrier()

    def body(g, _):
        pltpu.sync_copy(eidx_hbm.at[wid, pl.ds(pl.multiple_of(g * GRP, 8),
                                               GRP)], sbuf)
        for j in range(GRP):
            b = j % 2
            k = g * GRP + j
            pltpu.async_copy(h_hbm.at[sbuf.at[j]], rows_v.at[b], gsem.at[b])

            @pl.when(k >= 1)
            def _():
                pltpu.make_async_copy(rows_v.at[1 - b], acc_sh.at[idxd.at[0]],
                                      ssem.at[1 - b]).wait()
            pltpu.make_async_copy(h_hbm.at[sbuf.at[j]], rows_v.at[b],
                                  gsem.at[b]).wait()
            pltpu.async_copy(rows_v.at[b], acc_sh.at[idxd.at[k]],
                             ssem.at[b], add=True)
        return 0
    lax.fori_loop(0, cpt // GRP, body, 0)
    pltpu.make_async_copy(rows_v.at[1], acc_sh.at[idxd.at[0]],
                          ssem.at[1]).wait()
    plsc.subcore_barrier()

    sizes = [CHUNK] * 4 + [ROWS_PER_TILE - 4 * CHUNK]

    @pl.when(c == 0)
    def _():
        o = 0
        for sz in sizes:
            pltpu.sync_copy(acc_sh.at[pl.ds(base + o, sz)],
                            out_hbm.at[0, pl.ds(base + o, sz)])
            o += sz

    @pl.when(c == 1)
    def _():
        o = 0
        for sz in sizes:
            pltpu.sync_copy(acc_sh.at[pl.ds(base + o, sz)],
                            out_hbm.at[1, pl.ds(base + o, sz)])
            o += sz


def _make_rows_kernel(cpt):
    return pl.kernel(
        functools.partial(_rows_body, cpt),
        out_type=jax.ShapeDtypeStruct((2, N_ACC, D_FEAT), jnp.float32),
        mesh=_sc_mesh(),
        scratch_types=[
            pltpu.VMEM((cpt, CHUNK), jnp.int32),
            pltpu.VMEM((GRP, CHUNK), jnp.int32),
            pltpu.VMEM((2, CHUNK, D_FEAT), jnp.float32),
            pltpu.VMEM_SHARED((N_ACC, D_FEAT), jnp.float32),
            pltpu.SemaphoreType.DMA((2,)),
            pltpu.SemaphoreType.DMA((2,)),
        ],
    )


def _dinv(degp_ref, g):
    deg = degp_ref[g, 0, 0:N_NODE, :] + degp_ref[g, 1, 0:N_NODE, :] + 1.0
    return lax.rsqrt(jnp.maximum(deg, 1e-12))


def _mm_sub_body(x_ref, w_ref, degp_ref, o_ref):
    h = jnp.dot(x_ref[:], w_ref[:], preferred_element_type=jnp.float32)
    o_ref[0:N_NODE, :] = h * _dinv(degp_ref, 0)
    o_ref[N_NODE:N_ACC, :] = jnp.zeros((N_ACC - N_NODE, D_FEAT), jnp.float32)


def _fin_sub_body(hsub_ref, acc_ref, degp_ref, b_ref, batch_ref, o_ref):
    dinv = _dinv(degp_ref, 0)
    pre = (hsub_ref[0:N_NODE, :] + acc_ref[0, 0:N_NODE, :]
           + acc_ref[1, 0:N_NODE, :]) * dinv + b_ref[:]
    hs = jnp.maximum(pre, 0.0)
    onehot = (batch_ref[:] == lax.broadcasted_iota(
        jnp.int32, (N_NODE, N_BATCH), 1)).astype(jnp.float32)
    dn = (((0,), (0,)), ((), ()))
    psum = lax.dot_general(onehot, hs, dn,
                           preferred_element_type=jnp.float32)
    cnt = lax.dot_general(onehot, jnp.ones((N_NODE, 1), jnp.float32), dn,
                          preferred_element_type=jnp.float32)
    o_ref[:] = psum / jnp.maximum(cnt, 1.0)


def _mm_glob_body(x_ref, w_ref, degp_ref, pooled_ref, sidx_ref, o_ref):
    h = jnp.dot(x_ref[:], w_ref[:], preferred_element_type=jnp.float32)
    o_ref[0:N_NODE, :] = h
    for j in range(N_BATCH):
        r = (sidx_ref[j] - 1) % N_NODE
        xr = x_ref[pl.ds(r, 1), :] + pooled_ref[pl.ds(j, 1), :]
        o_ref[pl.ds(r, 1), :] = jnp.dot(xr, w_ref[:],
                                        preferred_element_type=jnp.float32)
    o_ref[0:N_NODE, :] = o_ref[0:N_NODE, :] * _dinv(degp_ref, 1)
    o_ref[N_NODE:N_ACC, :] = jnp.zeros((N_ACC - N_NODE, D_FEAT), jnp.float32)


def _fin_glob_body(hg_ref, acc_ref, degp_ref, b_ref, wfc_ref, bfc_ref, o_ref):
    dinv = _dinv(degp_ref, 1)
    pre = (hg_ref[0:N_NODE, :] + acc_ref[0, 0:N_NODE, :]
           + acc_ref[1, 0:N_NODE, :]) * dinv + b_ref[:]
    hg = jnp.maximum(pre, 0.0)
    emb = jnp.sum(hg, axis=0, keepdims=True) / jnp.float32(N_NODE)
    o_ref[:] = jnp.dot(emb, wfc_ref[:],
                       preferred_element_type=jnp.float32) + bfc_ref[:]


def _tc_call(body, out_shape, n_in, smem_args=()):
    in_specs = [pl.BlockSpec(memory_space=pltpu.VMEM) for _ in range(n_in)]
    for i in smem_args:
        in_specs[i] = pl.BlockSpec(memory_space=pltpu.SMEM)
    return pl.pallas_call(body, out_shape=out_shape, in_specs=in_specs)


def _round_up(x, m):
    return ((x + m - 1) // m) * m


@jax.jit
def kernel(sub_x, sub_edge_index, sub_batch, sub_index, global_x,
           global_edge_index, global_batch, W_sub, b_sub, W_glob, b_glob,
           W_fc, b_fc):
    e_sub = sub_edge_index.shape[1]
    e_glob = global_edge_index.shape[1]
    ep_sub = _round_up(e_sub, N_WORKERS * CHUNK * GRP)
    ep_glob = _round_up(e_glob, N_WORKERS * CHUNK * GRP)
    cpt_sub = ep_sub // (N_WORKERS * CHUNK)
    cpt_glob = ep_glob // (N_WORKERS * CHUNK)

    eidx_s = _pack_edges(sub_edge_index, ep_sub)
    eidx_g = _pack_edges(global_edge_index, ep_glob)
    zeros_blk = jnp.zeros((CHUNK, D_FEAT), jnp.float32)

    degp = _make_deg_kernel(cpt_sub, cpt_glob)(eidx_s, eidx_g)
    degp = degp.reshape(2, 2, N_ACC, 1)

    hsub = _tc_call(_mm_sub_body,
                    jax.ShapeDtypeStruct((N_ACC, D_FEAT), jnp.float32),
                    3)(sub_x, W_sub, degp)

    acc_s = _make_rows_kernel(cpt_sub)(hsub, eidx_s, zeros_blk)

    pooled = _tc_call(_fin_sub_body,
                      jax.ShapeDtypeStruct((N_BATCH, D_FEAT), jnp.float32),
                      5)(hsub, acc_s, degp, b_sub.reshape(1, D_FEAT),
                         sub_batch.reshape(N_NODE, 1))

    hglob = _tc_call(_mm_glob_body,
                     jax.ShapeDtypeStruct((N_ACC, D_FEAT), jnp.float32),
                     5, smem_args=(4,))(global_x, W_glob, degp, pooled,
                                        sub_index)

    acc_g = _make_rows_kernel(cpt_glob)(hglob, eidx_g, zeros_blk)

    out = _tc_call(_fin_glob_body,
                   jax.ShapeDtypeStruct((1, D_FEAT), jnp.float32),
                   6)(hglob, acc_g, degp, b_glob.reshape(1, D_FEAT),
                      W_fc, b_fc.reshape(1, D_FEAT))
    return out


# speedup vs baseline: 3.0389x; 1.0060x over previous
import functools

import jax
import jax.numpy as jnp
from jax import lax
from jax.experimental import pallas as pl
from jax.experimental.pallas import tpu as pltpu
from jax.experimental.pallas import tpu_sc as plsc

N_NODE = 10000
D_FEAT = 128
N_ACC = 10112
ROWS_PER_TILE = N_ACC // 16
PAD_IDX = 10000
N_BATCH = 16
CHUNK = 128
N_WORKERS = 32
GRP = 8


def _pack_edges(edge_index, e_pad):
    e = edge_index.shape[1]
    cpt = e_pad // (N_WORKERS * CHUNK)
    padv = PAD_IDX + jnp.arange(e_pad - e, dtype=jnp.int32) % (N_ACC - PAD_IDX)
    src = jnp.concatenate([edge_index[0], padv]).reshape(N_WORKERS, cpt, CHUNK)
    dst = jnp.concatenate([edge_index[1], padv]).reshape(N_WORKERS, cpt, CHUNK)
    return src, dst


def _sc_mesh():
    return plsc.VectorSubcoreMesh(core_axis_name="c", subcore_axis_name="s")


def _deg_body(cpt_sub, cpt_glob, dst_sub, dst_glob, out_hbm,
              idxd_s, idxd_g, ones_v, zbuf_v, deg_sub_sh, deg_glob_sh, sem):
    c = lax.axis_index("c")
    s = lax.axis_index("s")
    wid = c * 16 + s

    pltpu.sync_copy(dst_sub.at[wid], idxd_s)
    pltpu.sync_copy(dst_glob.at[wid], idxd_g)

    def fill(i, _):
        ones_v[pl.ds(i * 16, 16)] = jnp.ones((16,), jnp.float32)
        return 0
    lax.fori_loop(0, CHUNK // 16, fill, 0)

    def zfill(i, _):
        zbuf_v[pl.ds(i * 16, 16)] = jnp.zeros((16,), jnp.float32)
        return 0
    lax.fori_loop(0, N_ACC // 16, zfill, 0)

    @pl.when(s == 0)
    def _():
        pltpu.sync_copy(zbuf_v, deg_sub_sh)
        pltpu.sync_copy(zbuf_v, deg_glob_sh)
    plsc.subcore_barrier()

    def scatter_graph(idxd, deg_sh, cpt):
        def body(i, _):
            for j in range(8):
                pltpu.async_copy(ones_v, deg_sh.at[idxd.at[i * 8 + j]], sem,
                                 add=True)
            for j in range(8):
                pltpu.make_async_copy(ones_v, deg_sh.at[idxd.at[0]],
                                      sem).wait()
            return 0
        lax.fori_loop(0, cpt // 8, body, 0)

    scatter_graph(idxd_s, deg_sub_sh, cpt_sub)
    scatter_graph(idxd_g, deg_glob_sh, cpt_glob)
    plsc.subcore_barrier()

    @pl.when(jnp.logical_and(s == 0, c == 0))
    def _():
        pltpu.sync_copy(deg_sub_sh, out_hbm.at[0, 0])
        pltpu.sync_copy(deg_glob_sh, out_hbm.at[1, 0])

    @pl.when(jnp.logical_and(s == 0, c == 1))
    def _():
        pltpu.sync_copy(deg_sub_sh, out_hbm.at[0, 1])
        pltpu.sync_copy(deg_glob_sh, out_hbm.at[1, 1])


def _make_deg_kernel(cpt_sub, cpt_glob):
    return pl.kernel(
        functools.partial(_deg_body, cpt_sub, cpt_glob),
        out_type=jax.ShapeDtypeStruct((2, 2, N_ACC), jnp.float32),
        mesh=_sc_mesh(),
        scratch_types=[
            pltpu.VMEM((cpt_sub, CHUNK), jnp.int32),
            pltpu.VMEM((cpt_glob, CHUNK), jnp.int32),
            pltpu.VMEM((CHUNK,), jnp.float32),
            pltpu.VMEM((N_ACC,), jnp.float32),
            pltpu.VMEM_SHARED((N_ACC,), jnp.float32),
            pltpu.VMEM_SHARED((N_ACC,), jnp.float32),
            pltpu.SemaphoreType.DMA,
        ],
    )


def _rows_body(cpt, h_hbm, src_hbm, dst_hbm, zeros_hbm, out_hbm,
               idxd, sbuf, rows_v, acc_sh, gsem, ssem):
    c = lax.axis_index("c")
    s = lax.axis_index("s")
    wid = c * 16 + s

    pltpu.sync_copy(dst_hbm.at[wid], idxd)

    pltpu.sync_copy(zeros_hbm, rows_v.at[0])
    base = s * ROWS_PER_TILE
    for j in range(4):
        pltpu.sync_copy(rows_v.at[0], acc_sh.at[pl.ds(base + j * CHUNK, CHUNK)])
    pltpu.sync_copy(rows_v.at[0, pl.ds(0, ROWS_PER_TILE - 4 * CHUNK)],
                    acc_sh.at[pl.ds(base + 4 * CHUNK, ROWS_PER_TILE - 4 * CHUNK)])
    plsc.subcore_barrier()

    def body(g, _):
        pltpu.sync_copy(src_hbm.at[wid, pl.ds(pl.multiple_of(g * GRP, 8),
                                              GRP)], sbuf)
        for j in range(GRP):
            b = j % 2
            k = g * GRP + j
            pltpu.async_copy(h_hbm.at[sbuf.at[j]], rows_v.at[b], gsem.at[b])

            @pl.when(k >= 1)
            def _():
                pltpu.make_async_copy(rows_v.at[1 - b], acc_sh.at[idxd.at[0]],
                                      ssem.at[1 - b]).wait()
            pltpu.make_async_copy(h_hbm.at[sbuf.at[j]], rows_v.at[b],
                                  gsem.at[b]).wait()
            pltpu.async_copy(rows_v.at[b], acc_sh.at[idxd.at[k]],
                             ssem.at[b], add=True)
        return 0
    lax.fori_loop(0, cpt // GRP, body, 0)
    pltpu.make_async_copy(rows_v.at[1], acc_sh.at[idxd.at[0]],
                          ssem.at[1]).wait()
    plsc.subcore_barrier()

    sizes = [CHUNK] * 4 + [ROWS_PER_TILE - 4 * CHUNK]

    @pl.when(c == 0)
    def _():
        o = 0
        for sz in sizes:
            pltpu.sync_copy(acc_sh.at[pl.ds(base + o, sz)],
                            out_hbm.at[0, pl.ds(base + o, sz)])
            o += sz

    @pl.when(c == 1)
    def _():
        o = 0
        for sz in sizes:
            pltpu.sync_copy(acc_sh.at[pl.ds(base + o, sz)],
                            out_hbm.at[1, pl.ds(base + o, sz)])
            o += sz


def _make_rows_kernel(cpt):
    return pl.kernel(
        functools.partial(_rows_body, cpt),
        out_type=jax.ShapeDtypeStruct((2, N_ACC, D_FEAT), jnp.float32),
        mesh=_sc_mesh(),
        scratch_types=[
            pltpu.VMEM((cpt, CHUNK), jnp.int32),
            pltpu.VMEM((GRP, CHUNK), jnp.int32),
            pltpu.VMEM((2, CHUNK, D_FEAT), jnp.float32),
            pltpu.VMEM_SHARED((N_ACC, D_FEAT), jnp.float32),
            pltpu.SemaphoreType.DMA((2,)),
            pltpu.SemaphoreType.DMA((2,)),
        ],
    )


def _dinv(degp_ref, g):
    deg = degp_ref[g, 0, 0:N_NODE, :] + degp_ref[g, 1, 0:N_NODE, :] + 1.0
    return lax.rsqrt(jnp.maximum(deg, 1e-12))


def _mm_sub_body(x_ref, w_ref, degp_ref, o_ref):
    h = jnp.dot(x_ref[:], w_ref[:], preferred_element_type=jnp.float32)
    o_ref[0:N_NODE, :] = h * _dinv(degp_ref, 0)
    o_ref[N_NODE:N_ACC, :] = jnp.zeros((N_ACC - N_NODE, D_FEAT), jnp.float32)


def _fin_sub_body(hsub_ref, acc_ref, degp_ref, b_ref, batch_ref, o_ref):
    dinv = _dinv(degp_ref, 0)
    pre = (hsub_ref[0:N_NODE, :] + acc_ref[0, 0:N_NODE, :]
           + acc_ref[1, 0:N_NODE, :]) * dinv + b_ref[:]
    hs = jnp.maximum(pre, 0.0)
    onehot = (batch_ref[:] == lax.broadcasted_iota(
        jnp.int32, (N_NODE, N_BATCH), 1)).astype(jnp.float32)
    dn = (((0,), (0,)), ((), ()))
    psum = lax.dot_general(onehot, hs, dn,
                           preferred_element_type=jnp.float32)
    cnt = lax.dot_general(onehot, jnp.ones((N_NODE, 1), jnp.float32), dn,
                          preferred_element_type=jnp.float32)
    o_ref[:] = psum / jnp.maximum(cnt, 1.0)


def _mm_glob_body(x_ref, w_ref, degp_ref, pooled_ref, sidx_ref, o_ref):
    h = jnp.dot(x_ref[:], w_ref[:], preferred_element_type=jnp.float32)
    o_ref[0:N_NODE, :] = h
    for j in range(N_BATCH):
        r = (sidx_ref[j] - 1) % N_NODE
        xr = x_ref[pl.ds(r, 1), :] + pooled_ref[pl.ds(j, 1), :]
        o_ref[pl.ds(r, 1), :] = jnp.dot(xr, w_ref[:],
                                        preferred_element_type=jnp.float32)
    o_ref[0:N_NODE, :] = o_ref[0:N_NODE, :] * _dinv(degp_ref, 1)
    o_ref[N_NODE:N_ACC, :] = jnp.zeros((N_ACC - N_NODE, D_FEAT), jnp.float32)


def _fin_glob_body(hg_ref, acc_ref, degp_ref, b_ref, wfc_ref, bfc_ref, o_ref):
    dinv = _dinv(degp_ref, 1)
    pre = (hg_ref[0:N_NODE, :] + acc_ref[0, 0:N_NODE, :]
           + acc_ref[1, 0:N_NODE, :]) * dinv + b_ref[:]
    hg = jnp.maximum(pre, 0.0)
    emb = jnp.sum(hg, axis=0, keepdims=True) / jnp.float32(N_NODE)
    o_ref[:] = jnp.dot(emb, wfc_ref[:],
                       preferred_element_type=jnp.float32) + bfc_ref[:]


def _tc_call(body, out_shape, n_in, smem_args=()):
    in_specs = [pl.BlockSpec(memory_space=pltpu.VMEM) for _ in range(n_in)]
    for i in smem_args:
        in_specs[i] = pl.BlockSpec(memory_space=pltpu.SMEM)
    return pl.pallas_call(body, out_shape=out_shape, in_specs=in_specs)


def _round_up(x, m):
    return ((x + m - 1) // m) * m


@jax.jit
def kernel(sub_x, sub_edge_index, sub_batch, sub_index, global_x,
           global_edge_index, global_batch, W_sub, b_sub, W_glob, b_glob,
           W_fc, b_fc):
    e_sub = sub_edge_index.shape[1]
    e_glob = global_edge_index.shape[1]
    ep_sub = _round_up(e_sub, N_WORKERS * CHUNK * GRP)
    ep_glob = _round_up(e_glob, N_WORKERS * CHUNK * GRP)
    cpt_sub = ep_sub // (N_WORKERS * CHUNK)
    cpt_glob = ep_glob // (N_WORKERS * CHUNK)

    src_s, dst_s = _pack_edges(sub_edge_index, ep_sub)
    src_g, dst_g = _pack_edges(global_edge_index, ep_glob)
    zeros_blk = jnp.zeros((CHUNK, D_FEAT), jnp.float32)

    degp = _make_deg_kernel(cpt_sub, cpt_glob)(dst_s, dst_g)
    degp = degp.reshape(2, 2, N_ACC, 1)

    hsub = _tc_call(_mm_sub_body,
                    jax.ShapeDtypeStruct((N_ACC, D_FEAT), jnp.float32),
                    3)(sub_x, W_sub, degp)

    acc_s = _make_rows_kernel(cpt_sub)(hsub, src_s, dst_s, zeros_blk)

    pooled = _tc_call(_fin_sub_body,
                      jax.ShapeDtypeStruct((N_BATCH, D_FEAT), jnp.float32),
                      5)(hsub, acc_s, degp, b_sub.reshape(1, D_FEAT),
                         sub_batch.reshape(N_NODE, 1))

    hglob = _tc_call(_mm_glob_body,
                     jax.ShapeDtypeStruct((N_ACC, D_FEAT), jnp.float32),
                     5, smem_args=(4,))(global_x, W_glob, degp, pooled,
                                        sub_index)

    acc_g = _make_rows_kernel(cpt_glob)(hglob, src_g, dst_g, zeros_blk)

    out = _tc_call(_fin_glob_body,
                   jax.ShapeDtypeStruct((1, D_FEAT), jnp.float32),
                   6)(hglob, acc_g, degp, b_glob.reshape(1, D_FEAT),
                      W_fc, b_fc.reshape(1, D_FEAT))
    return out
